# baseline (speedup 1.0000x reference)
import functools

import jax
import jax.numpy as jnp
from jax import lax
from jax.experimental import pallas as pl
from jax.experimental.pallas import tpu as pltpu
from jax.experimental.pallas import tpu_sc as plsc

_B = 16384
_D = 32
_NC = 2
_NS = 16
_NW = _NC * _NS
_BPW = _B // _NW
_CHUNK = 128
_NCHUNK = _BPW // _CHUNK

_TC_BLOCK = 2048


def _sc_gather_body(uidx_hbm, iidx_hbm, utab_hbm, itab_hbm,
                    u_out, i_out,
                    uidx_v, iidx_v, urows_v, irows_v, sem):
    wid = lax.axis_index("s") * _NC + lax.axis_index("c")
    base = wid * _BPW
    pltpu.sync_copy(uidx_hbm.at[pl.ds(wid * _NCHUNK, _NCHUNK)], uidx_v)
    pltpu.sync_copy(iidx_hbm.at[pl.ds(wid * _NCHUNK, _NCHUNK)], iidx_v)
    copies = []
    for j in range(_NCHUNK):
        copies.append(pltpu.async_copy(
            utab_hbm.at[uidx_v.at[j]],
            urows_v.at[pl.ds(j * _CHUNK, _CHUNK)], sem))
        copies.append(pltpu.async_copy(
            itab_hbm.at[iidx_v.at[j]],
            irows_v.at[pl.ds(j * _CHUNK, _CHUNK)], sem))
    for c in copies:
        c.wait()
    pltpu.sync_copy(urows_v, u_out.at[pl.ds(base, _BPW)])
    pltpu.sync_copy(irows_v, i_out.at[pl.ds(base, _BPW)])


def _sc_gather(user_idx, item_idx, user_emb, item_emb):
    mesh = plsc.VectorSubcoreMesh(core_axis_name="c", subcore_axis_name="s")
    f = pl.kernel(
        _sc_gather_body,
        mesh=mesh,
        compiler_params=pltpu.CompilerParams(use_tc_tiling_on_sc=False),
        out_type=(
            jax.ShapeDtypeStruct((_B, _D), jnp.float32),
            jax.ShapeDtypeStruct((_B, _D), jnp.float32),
        ),
        scratch_types=[
            pltpu.VMEM((_NCHUNK, _CHUNK), jnp.int32),
            pltpu.VMEM((_NCHUNK, _CHUNK), jnp.int32),
            pltpu.VMEM((_BPW, _D), jnp.float32),
            pltpu.VMEM((_BPW, _D), jnp.float32),
            pltpu.SemaphoreType.DMA,
        ],
    )
    uidx2 = user_idx.reshape(_NW * _NCHUNK, _CHUNK)
    iidx2 = item_idx.reshape(_NW * _NCHUNK, _CHUNK)
    return f(uidx2, iidx2, user_emb, item_emb)


def _tc_body(u_ref, i_ref, dn_ref, Wd_ref, bd_ref,
             W1u_ref, W1i_ref, W1d_ref, b1_ref,
             W2_ref, b2_ref, W3_ref, b3_ref, out_ref):
    u = u_ref[...]
    it = i_ref[...]
    dn = dn_ref[...]
    d = jnp.dot(dn, Wd_ref[...], preferred_element_type=jnp.float32) + bd_ref[...]
    s = u + it + d
    fm = 0.5 * jnp.sum(s * s - u * u - it * it - d * d, axis=1, keepdims=True)
    h = (jnp.dot(u, W1u_ref[...], preferred_element_type=jnp.float32)
         + jnp.dot(it, W1i_ref[...], preferred_element_type=jnp.float32)
         + jnp.dot(dn, W1d_ref[...], preferred_element_type=jnp.float32)
         + b1_ref[...])
    h = jnp.maximum(h, 0.0)
    h = jnp.maximum(
        jnp.dot(h, W2_ref[...], preferred_element_type=jnp.float32) + b2_ref[...],
        0.0)
    out = jnp.dot(h, W3_ref[...], preferred_element_type=jnp.float32) + b3_ref[...]
    out_ref[...] = out + fm


def _tc_compute(u, i, dense, Wd, bd, W1, b1, W2, b2, W3, b3):
    nd = dense.shape[1]
    h1 = W1.shape[1]
    h2 = W2.shape[1]
    W1u = W1[:_D]
    W1i = W1[_D:2 * _D]
    W1d = W1[2 * _D:]
    grid = _B // _TC_BLOCK

    def batch_spec(cols):
        return pl.BlockSpec((_TC_BLOCK, cols), lambda b: (b, 0))

    def full_spec(shape):
        return pl.BlockSpec(shape, lambda b: (0,) * len(shape))

    out = pl.pallas_call(
        _tc_body,
        grid=(grid,),
        in_specs=[
            batch_spec(_D), batch_spec(_D), batch_spec(nd),
            full_spec(Wd.shape), full_spec((1, _D)),
            full_spec(W1u.shape), full_spec(W1i.shape), full_spec(W1d.shape),
            full_spec((1, h1)),
            full_spec(W2.shape), full_spec((1, h2)),
            full_spec(W3.shape), full_spec((1, 1)),
        ],
        out_specs=pl.BlockSpec((_TC_BLOCK, 1), lambda b: (b, 0)),
        out_shape=jax.ShapeDtypeStruct((_B, 1), jnp.float32),
    )(u, i, dense, Wd, bd.reshape(1, _D),
      W1u, W1i, W1d, b1.reshape(1, h1),
      W2, b2.reshape(1, h2), W3, b3.reshape(1, 1))
    return out[:, 0]


def kernel(user_idx, item_idx, dense, user_emb, item_emb,
           Wd, bd, W1, b1, W2, b2, W3, b3):
    u, i = _sc_gather(user_idx, item_idx, user_emb, item_emb)
    return _tc_compute(u, i, dense, Wd, bd, W1, b1, W2, b2, W3, b3)
